# baseline (speedup 1.0000x reference)
import functools

import jax
import jax.numpy as jnp
from jax.experimental import pallas as pl
from jax.experimental.pallas import tpu as pltpu


def _gcn_kernel(x_ref, a_ref, w_ref, out_ref):
    a_bf = a_ref[...].astype(jnp.bfloat16)
    t = jnp.dot(a_bf, x_ref[...], preferred_element_type=jnp.float32)
    acc = jnp.dot(t, w_ref[...], preferred_element_type=jnp.float32)
    out_ref[...] = jnp.maximum(acc, 0.0)


@functools.partial(jax.jit, static_argnames=("block_m",))
def _gcn(x, a, conv_w, block_m):
    n, in_dim = x.shape
    out_dim = conv_w.shape[1]
    num_blocks = pl.cdiv(a.shape[0], block_m)
    return pl.pallas_call(
        _gcn_kernel,
        grid=(num_blocks,),
        in_specs=[
            pl.BlockSpec((n, in_dim), lambda i: (0, 0)),
            pl.BlockSpec((block_m, n), lambda i: (i, 0)),
            pl.BlockSpec((in_dim, out_dim), lambda i: (0, 0)),
        ],
        out_specs=pl.BlockSpec((block_m, out_dim), lambda i: (i, 0)),
        out_shape=jax.ShapeDtypeStruct((a.shape[0], out_dim), jnp.float32),
        compiler_params=pltpu.CompilerParams(
            dimension_semantics=("arbitrary",),
        ),
    )(x, a, conv_w)


def kernel(x, a, conv_w):
    x = x.astype(jnp.bfloat16)
    block_m = 400 if a.shape[0] % 400 == 0 else a.shape[0]
    return _gcn(x, a, conv_w, block_m)


# speedup vs baseline: 1.0226x; 1.0226x over previous
import functools

import jax
import jax.numpy as jnp
from jax.experimental import pallas as pl
from jax.experimental.pallas import tpu as pltpu


def _gcn_kernel(x_ref, a_ref, w_ref, out_ref, hidden_ref):
    i = pl.program_id(0)

    @pl.when(i == 0)
    def _():
        hidden_ref[...] = jnp.dot(
            x_ref[...], w_ref[...], preferred_element_type=jnp.float32
        )

    acc = jnp.dot(a_ref[...], hidden_ref[...], preferred_element_type=jnp.float32)
    out_ref[...] = jnp.maximum(acc, 0.0)


@functools.partial(jax.jit, static_argnames=("block_m",))
def _gcn(x, a, conv_w, block_m):
    n, in_dim = x.shape
    out_dim = conv_w.shape[1]
    num_blocks = pl.cdiv(a.shape[0], block_m)
    return pl.pallas_call(
        _gcn_kernel,
        grid=(num_blocks,),
        in_specs=[
            pl.BlockSpec((n, in_dim), lambda i: (0, 0)),
            pl.BlockSpec((block_m, n), lambda i: (i, 0)),
            pl.BlockSpec((in_dim, out_dim), lambda i: (0, 0)),
        ],
        out_specs=pl.BlockSpec((block_m, out_dim), lambda i: (i, 0)),
        out_shape=jax.ShapeDtypeStruct((a.shape[0], out_dim), jnp.float32),
        scratch_shapes=[pltpu.VMEM((n, out_dim), jnp.float32)],
        compiler_params=pltpu.CompilerParams(
            dimension_semantics=("arbitrary",),
        ),
    )(x, a, conv_w)


def kernel(x, a, conv_w):
    x = x.astype(jnp.float32)
    block_m = 400 if a.shape[0] % 400 == 0 else a.shape[0]
    return _gcn(x, a, conv_w, block_m)
